# baseline (speedup 1.0000x reference)
import functools

import jax
import jax.numpy as jnp
from jax import lax
from jax.experimental import pallas as pl
from jax.experimental.pallas import tpu as pltpu
from jax.experimental.pallas import tpu_sc as plsc

ND, NG, D, E, DO = 10000, 50000, 128, 600000, 128
NC, NS, L = 2, 16, 16
NW = NC * NS
G = 128
NGROUPS = 4704
EP = NGROUPS * G
GPT = NGROUPS // NW
GPS = NGROUPS // NS
ND_PAD = 10112
ND_ROWS = ND_PAD // NS
NG_R = 8448
NG_ACC = 8576
NG_ROWS = NG_ACC // NS

_mesh = plsc.VectorSubcoreMesh(core_axis_name="c", subcore_axis_name="s")


CH_D = 16
CHUNKS_D = (16,) * 9 + (3,)

@functools.partial(
    pl.kernel, mesh=_mesh,
    compiler_params=pltpu.CompilerParams(needs_layout_passes=False),
    out_type=[jax.ShapeDtypeStruct((NC, ND_PAD, D), jnp.float32),
              jax.ShapeDtypeStruct((NW, ND_PAD), jnp.float32)],
    scratch_types=[
        pltpu.VMEM((CH_D, G), jnp.int32),
        pltpu.VMEM((CH_D, G), jnp.int32),
        pltpu.VMEM((2, G, D), jnp.float32),
        pltpu.VMEM((ND_PAD,), jnp.float32),
        pltpu.VMEM_SHARED((ND_PAD, D), jnp.float32),
        pltpu.SemaphoreType.DMA,
        pltpu.SemaphoreType.DMA,
    ])
def _sc_disease(xg, src2, dst2, zrow, zvec, agg_out, cnt_out,
                gidx, sidx, rows, cntv, acc, sem0, sem1):
    cid = lax.axis_index("c")
    sid = lax.axis_index("s")
    w = cid * NS + sid
    pltpu.sync_copy(zrow.at[pl.ds(0, ND_ROWS)], acc.at[pl.ds(sid * ND_ROWS, ND_ROWS)])
    pltpu.sync_copy(zvec.at[pl.ds(0, ND_PAD)], cntv)
    ones16 = jnp.ones((L,), jnp.float32)
    plsc.subcore_barrier()

    def drain(j, b):
        pltpu.make_async_copy(xg.at[gidx.at[0]], rows.at[b],
                              sem0 if b == 0 else sem1).wait()
        pltpu.sync_copy(rows.at[b], acc.at[sidx.at[j]], add=True)
        for k in range(8):
            plsc.addupdate_scatter(cntv, [sidx[j, pl.ds(k * L, L)]], ones16)

    for ch, clen in enumerate(CHUNKS_D):
        pltpu.sync_copy(dst2.at[w, pl.ds(ch * CH_D, clen)], gidx.at[pl.ds(0, clen)])
        pltpu.sync_copy(src2.at[w, pl.ds(ch * CH_D, clen)], sidx.at[pl.ds(0, clen)])
        pltpu.async_copy(xg.at[gidx.at[0]], rows.at[0], sem0)

        def body(j, carry):
            @pl.when(j % 2 == 0)
            def _():
                @pl.when(j + 1 < clen)
                def _():
                    pltpu.async_copy(xg.at[gidx.at[j + 1]], rows.at[1], sem1)
                drain(j, 0)

            @pl.when(j % 2 == 1)
            def _():
                @pl.when(j + 1 < clen)
                def _():
                    pltpu.async_copy(xg.at[gidx.at[j + 1]], rows.at[0], sem0)
                drain(j, 1)

            return carry

        lax.fori_loop(0, clen, body, 0)

    plsc.subcore_barrier()
    pltpu.sync_copy(acc.at[pl.ds(sid * ND_ROWS, ND_ROWS)],
                    agg_out.at[cid, pl.ds(sid * ND_ROWS, ND_ROWS)])
    pltpu.sync_copy(cntv, cnt_out.at[w])


CH_G = 24
CHUNKS_G = (24,) * 12 + (6,)

@functools.partial(
    pl.kernel, mesh=_mesh,
    compiler_params=pltpu.CompilerParams(needs_layout_passes=False),
    out_type=[jax.ShapeDtypeStruct((NC, 3, NG_ACC, D), jnp.float32),
              jax.ShapeDtypeStruct((NC, 3, NS, NG_ACC), jnp.float32)],
    scratch_types=[
        pltpu.VMEM((CH_G, G), jnp.int32),
        pltpu.VMEM((CH_G, G), jnp.int32),
        pltpu.VMEM((4, G), jnp.int32),
        pltpu.VMEM((4, G), jnp.int32),
        pltpu.VMEM((2, G, D), jnp.float32),
        pltpu.VMEM((NG_ACC,), jnp.float32),
        pltpu.VMEM_SHARED((NG_ACC, D), jnp.float32),
        pltpu.SemaphoreType.DMA,
        pltpu.SemaphoreType.DMA,
    ])
def _sc_gene(xd, src2, dst2, zrow, zvec, agg_out, cnt_out,
             ebuf_s, ebuf_d, ring_s, ring_d, rows, cntv, acc, sem0, sem1):
    cid = lax.axis_index("c")
    sid = lax.axis_index("s")
    iota = lax.iota(jnp.int32, L)
    zero16 = jnp.zeros((L,), jnp.int32)
    trash16 = jnp.full((L,), NG_R, jnp.int32)
    ones16 = jnp.ones((L,), jnp.float32)

    ones16 = jnp.ones((L,), jnp.float32)

    def drain(q):
        sem = sem0 if q & 1 == 0 else sem1
        pltpu.make_async_copy(xd.at[ring_s.at[q]], rows.at[q & 1], sem).wait()
        pltpu.sync_copy(rows.at[q & 1], acc.at[ring_d.at[q]], add=True)
        for k in range(8):
            plsc.addupdate_scatter(cntv, [ring_d[q, pl.ds(k * L, L)]], ones16)

    def start(q):
        sem = sem0 if q & 1 == 0 else sem1
        pltpu.async_copy(xd.at[ring_s.at[q]], rows.at[q & 1], sem)

    def drain4(qd):
        for q in range(4):
            @pl.when(qd == q)
            def _():
                drain(q)

    def start4(qd):
        for q in range(4):
            @pl.when(qd == q)
            def _():
                start(q)

    def pass_body(p, pcarry):
        base = (2 * p + cid) * NG_R
        pltpu.sync_copy(zrow.at[pl.ds(0, NG_ROWS)],
                        acc.at[pl.ds(sid * NG_ROWS, NG_ROWS)])
        pltpu.sync_copy(zvec.at[pl.ds(0, NG_ACC)], cntv)
        for k in range(8):
            ring_s[3, pl.ds(k * L, L)] = zero16
            ring_d[3, pl.ds(k * L, L)] = trash16
        start(3)
        plsc.subcore_barrier()

        n = jnp.int32(0)
        for ch, clen in enumerate(CHUNKS_G):
            pltpu.sync_copy(src2.at[sid, pl.ds(ch * CH_G, clen)],
                            ebuf_s.at[pl.ds(0, clen)])
            pltpu.sync_copy(dst2.at[sid, pl.ds(ch * CH_G, clen)],
                            ebuf_d.at[pl.ds(0, clen)])

            def scan_group(j, n):
                n0 = n
                for k in range(8):
                    dv = ebuf_d[j, pl.ds(k * L, L)]
                    sv = ebuf_s[j, pl.ds(k * L, L)]
                    m = (dv >= base) & (dv < base + NG_R)
                    cum = plsc.cumsum(m.astype(jnp.int32))
                    pos = n + cum - 1
                    plsc.store_scatter(ring_s, [(pos >> 7) & 3, pos & 127], sv, mask=m)
                    plsc.store_scatter(ring_d, [(pos >> 7) & 3, pos & 127], dv - base,
                                       mask=m)
                    n = n + cum[L - 1]

                @pl.when((n >> 7) != (n0 >> 7))
                def _():
                    g = n0 >> 7
                    drain4((g - 1) & 3)
                    start4(g & 3)

                return n

            n = lax.fori_loop(0, clen, scan_group, n)

        @pl.when((n & 127) != 0)
        def _():
            g = n >> 7
            drain4((g - 1) & 3)
            for k in range(8):
                pp = n + k * L + iota
                plsc.store_scatter(ring_d, [(pp >> 7) & 3, pp & 127], trash16)
                plsc.store_scatter(ring_s, [(pp >> 7) & 3, pp & 127], zero16)
            start4(g & 3)

        drain4((((n + G - 1) >> 7) - 1) & 3)

        plsc.subcore_barrier()
        pltpu.sync_copy(acc.at[pl.ds(sid * NG_ROWS, NG_ROWS)],
                        agg_out.at[cid, p, pl.ds(sid * NG_ROWS, NG_ROWS)])
        pltpu.sync_copy(cntv, cnt_out.at[cid, p, sid])
        plsc.subcore_barrier()
        return pcarry

    lax.fori_loop(0, 3, pass_body, 0)


@functools.partial(
    pl.kernel, mesh=_mesh,
    compiler_params=pltpu.CompilerParams(needs_layout_passes=False),
    out_type=jax.ShapeDtypeStruct((EP,), jnp.float32),
    scratch_types=[
        pltpu.VMEM((GPT, G), jnp.int32),
        pltpu.VMEM((GPT, G), jnp.int32),
        pltpu.VMEM((2, G, D), jnp.float32),
        pltpu.VMEM((2, G, D), jnp.float32),
        pltpu.VMEM((G,), jnp.float32),
        pltpu.SemaphoreType.DMA,
        pltpu.SemaphoreType.DMA,
    ])
def _sc_decoder(zd, zg, src2, dst2, out, idx_s, idx_d, rows_d, rows_g, ob,
                sem0, sem1):
    cid = lax.axis_index("c")
    sid = lax.axis_index("s")
    w = cid * NS + sid
    pltpu.sync_copy(src2.at[w], idx_s)
    pltpu.sync_copy(dst2.at[w], idx_d)
    iota = lax.iota(jnp.int32, L)
    last = iota == (L - 1)

    def start(j, b, sem):
        pltpu.async_copy(zd.at[idx_s.at[j]], rows_d.at[b], sem)
        pltpu.async_copy(zg.at[idx_d.at[j]], rows_g.at[b], sem)

    def wait(b, sem):
        pltpu.make_async_copy(zd.at[idx_s.at[0]], rows_d.at[b], sem).wait()
        pltpu.make_async_copy(zg.at[idx_d.at[0]], rows_g.at[b], sem).wait()

    def compute(j, b):
        def dot_one(e, carry):
            acc = rows_d[b, e, pl.ds(0, L)] * rows_g[b, e, pl.ds(0, L)]
            for k in range(1, 8):
                acc = acc + rows_d[b, e, pl.ds(k * L, L)] * rows_g[b, e, pl.ds(k * L, L)]
            cum = plsc.cumsum(acc)
            plsc.store_scatter(ob, [jnp.full((L,), e, jnp.int32)], cum, mask=last)
            return carry

        lax.fori_loop(0, G, dot_one, 0)
        for k in range(8):
            v = ob[pl.ds(k * L, L)]
            ob[pl.ds(k * L, L)] = 1.0 / (1.0 + jnp.exp(-v))
        pltpu.sync_copy(ob, out.at[pl.ds((w * GPT + j) * G, G)])

    start(0, 0, sem0)

    def body(j, carry):
        @pl.when(j % 2 == 0)
        def _():
            wait(0, sem0)

            @pl.when(j + 1 < GPT)
            def _():
                start(j + 1, 1, sem1)

            compute(j, 0)

        @pl.when(j % 2 == 1)
        def _():
            wait(1, sem1)

            @pl.when(j + 1 < GPT)
            def _():
                start(j + 1, 0, sem0)

            compute(j, 1)

        return carry

    lax.fori_loop(0, GPT, body, 0)


_BLK = 1000


def _dense_body(npart, *refs):
    aggs = refs[:npart]
    cnt_ref = refs[npart]
    (x_ref, wl_ref, wr_ref, b_ref, wmu_ref, bmu_ref, wlv_ref, blv_ref,
     eps_ref, z_ref) = refs[npart + 1:]
    agg = aggs[0][...]
    for a in aggs[1:]:
        agg = agg + a[...]
    cnt = jnp.sum(cnt_ref[...], axis=1, keepdims=True)
    mean = agg / jnp.maximum(cnt, 1.0)
    h = (jnp.dot(mean, wl_ref[...], preferred_element_type=jnp.float32)
         + jnp.dot(x_ref[...], wr_ref[...], preferred_element_type=jnp.float32)
         + b_ref[...][None, :])
    mu = jnp.dot(h, wmu_ref[...], preferred_element_type=jnp.float32) + bmu_ref[...][None, :]
    lv = jnp.minimum(
        jnp.dot(h, wlv_ref[...], preferred_element_type=jnp.float32) + blv_ref[...][None, :],
        10.0)
    z_ref[...] = mu + eps_ref[...] * jnp.exp(lv)


def _dense_stage(aggs, cnt, x, W_l, W_r, b, W_mu, b_mu, W_lv, b_lv, eps):
    n = x.shape[0]
    npart = len(aggs)
    ncnt = cnt.shape[1]
    row_spec = pl.BlockSpec((_BLK, D), lambda i: (i, 0))
    cnt_spec = pl.BlockSpec((_BLK, ncnt), lambda i: (i, 0))
    w_spec = pl.BlockSpec((D, DO), lambda i: (0, 0))
    b_spec = pl.BlockSpec((DO,), lambda i: (0,))
    return pl.pallas_call(
        functools.partial(_dense_body, npart),
        grid=(n // _BLK,),
        in_specs=([row_spec] * npart + [cnt_spec]
                  + [row_spec, w_spec, w_spec, b_spec, w_spec, b_spec,
                     w_spec, b_spec, row_spec]),
        out_specs=pl.BlockSpec((_BLK, DO), lambda i: (i, 0)),
        out_shape=jax.ShapeDtypeStruct((n, DO), jnp.float32),
    )(*aggs, cnt, x, W_l, W_r, b, W_mu, b_mu, W_lv, b_lv, eps)


def kernel(x_disease, x_gene, edge_src, edge_dst, W_l, W_r, b_sage,
           W_mu_d, b_mu_d, W_lv_d, b_lv_d, W_mu_g, b_mu_g, W_lv_g, b_lv_g):
    ek = jax.random.split(jax.random.key(42), 2)
    eps_d = jax.random.normal(ek[0], (ND, DO), dtype=jnp.float32)
    eps_g = jax.random.normal(ek[1], (NG, DO), dtype=jnp.float32)

    pad_e = EP - E
    srcp = jnp.concatenate([edge_src, jnp.full((pad_e,), ND, jnp.int32)])
    dstp = jnp.concatenate([edge_dst, jnp.full((pad_e,), NG, jnp.int32)])
    src32 = srcp.reshape(NW, GPT, G)
    dst32 = dstp.reshape(NW, GPT, G)
    src16 = srcp.reshape(NS, GPS, G)
    dst16 = dstp.reshape(NS, GPS, G)
    xd_pad = jnp.concatenate([x_disease, jnp.zeros((ND_PAD - ND, D), jnp.float32)])
    xg_pad = jnp.concatenate([x_gene, jnp.zeros((16, D), jnp.float32)])

    zr = max(ND_ROWS, NG_ROWS)
    zrow = jnp.zeros((zr, D), jnp.float32)
    zvec = jnp.zeros((max(ND_PAD, NG_ACC),), jnp.float32)

    agg_d_p, cnt_d_p = _sc_disease(xg_pad, src32, dst32, zrow, zvec)
    agg_g_p, cnt_g_p = _sc_gene(xd_pad, src16, dst16, zrow, zvec)

    sizes = [NG_R] * 5 + [NG - 5 * NG_R]
    agg_g = jnp.concatenate([agg_g_p[r % 2, r // 2, :sizes[r]] for r in range(6)])
    cnt_g = jnp.transpose(
        jnp.concatenate([cnt_g_p[r % 2, r // 2, :, :sizes[r]] for r in range(6)],
                        axis=1))
    cnt_d = jnp.transpose(cnt_d_p)[:ND]

    z_g = _dense_stage([agg_g], cnt_g, x_gene, W_l, W_r, b_sage,
                       W_mu_g, b_mu_g, W_lv_g, b_lv_g, eps_g)
    z_d = _dense_stage([agg_d_p[0, :ND], agg_d_p[1, :ND]],
                       cnt_d,
                       x_disease, W_l, W_r, b_sage,
                       W_mu_d, b_mu_d, W_lv_d, b_lv_d, eps_d)

    zd_pad = jnp.concatenate([z_d, jnp.zeros((ND_PAD - ND, DO), jnp.float32)])
    zg_pad = jnp.concatenate([z_g, jnp.zeros((16, DO), jnp.float32)])
    out = _sc_decoder(zd_pad, zg_pad, src32, dst32)
    return out[:E]


# speedup vs baseline: 1.0083x; 1.0083x over previous
import functools

import jax
import jax.numpy as jnp
from jax import lax
from jax.experimental import pallas as pl
from jax.experimental.pallas import tpu as pltpu
from jax.experimental.pallas import tpu_sc as plsc

ND, NG, D, E, DO = 10000, 50000, 128, 600000, 128
NC, NS, L = 2, 16, 16
NW = NC * NS
G = 128
NGROUPS = 4704
EP = NGROUPS * G
GPT = NGROUPS // NW
GPS = NGROUPS // NS
ND_PAD = 10112
ND_ROWS = ND_PAD // NS
NG_R = 8448
NG_ACC = 8576
NG_ROWS = NG_ACC // NS

_mesh = plsc.VectorSubcoreMesh(core_axis_name="c", subcore_axis_name="s")


CH_D = 16
CHUNKS_D = (16,) * 9 + (3,)

@functools.partial(
    pl.kernel, mesh=_mesh,
    compiler_params=pltpu.CompilerParams(needs_layout_passes=False),
    out_type=[jax.ShapeDtypeStruct((NC, ND_PAD, D), jnp.float32),
              jax.ShapeDtypeStruct((NW, ND_PAD), jnp.float32)],
    scratch_types=[
        pltpu.VMEM((CH_D, G), jnp.int32),
        pltpu.VMEM((CH_D, G), jnp.int32),
        pltpu.VMEM((2, G, D), jnp.float32),
        pltpu.VMEM((ND_PAD,), jnp.float32),
        pltpu.VMEM_SHARED((ND_PAD, D), jnp.float32),
        pltpu.SemaphoreType.DMA,
        pltpu.SemaphoreType.DMA,
    ])
def _sc_disease(xg, src2, dst2, zrow, zvec, agg_out, cnt_out,
                gidx, sidx, rows, cntv, acc, sem0, sem1):
    cid = lax.axis_index("c")
    sid = lax.axis_index("s")
    w = cid * NS + sid
    pltpu.sync_copy(zrow.at[pl.ds(0, ND_ROWS)], acc.at[pl.ds(sid * ND_ROWS, ND_ROWS)])
    pltpu.sync_copy(zvec.at[pl.ds(0, ND_PAD)], cntv)
    ones16 = jnp.ones((L,), jnp.float32)
    plsc.subcore_barrier()

    def drain(j, b):
        pltpu.make_async_copy(xg.at[gidx.at[0]], rows.at[b],
                              sem0 if b == 0 else sem1).wait()
        pltpu.sync_copy(rows.at[b], acc.at[sidx.at[j]], add=True)
        for k in range(8):
            plsc.addupdate_scatter(cntv, [sidx[j, pl.ds(k * L, L)]], ones16)

    for ch, clen in enumerate(CHUNKS_D):
        pltpu.sync_copy(dst2.at[w, pl.ds(ch * CH_D, clen)], gidx.at[pl.ds(0, clen)])
        pltpu.sync_copy(src2.at[w, pl.ds(ch * CH_D, clen)], sidx.at[pl.ds(0, clen)])
        pltpu.async_copy(xg.at[gidx.at[0]], rows.at[0], sem0)

        def body(j, carry):
            @pl.when(j % 2 == 0)
            def _():
                @pl.when(j + 1 < clen)
                def _():
                    pltpu.async_copy(xg.at[gidx.at[j + 1]], rows.at[1], sem1)
                drain(j, 0)

            @pl.when(j % 2 == 1)
            def _():
                @pl.when(j + 1 < clen)
                def _():
                    pltpu.async_copy(xg.at[gidx.at[j + 1]], rows.at[0], sem0)
                drain(j, 1)

            return carry

        lax.fori_loop(0, clen, body, 0)

    plsc.subcore_barrier()
    pltpu.sync_copy(acc.at[pl.ds(sid * ND_ROWS, ND_ROWS)],
                    agg_out.at[cid, pl.ds(sid * ND_ROWS, ND_ROWS)])
    pltpu.sync_copy(cntv, cnt_out.at[w])


CH_G = 24
CHUNKS_G = (24,) * 12 + (6,)

@functools.partial(
    pl.kernel, mesh=_mesh,
    compiler_params=pltpu.CompilerParams(needs_layout_passes=False),
    out_type=[jax.ShapeDtypeStruct((NC, 3, NG_ACC, D), jnp.float32),
              jax.ShapeDtypeStruct((NC, 3, NS, NG_ACC), jnp.float32)],
    scratch_types=[
        pltpu.VMEM((CH_G, G), jnp.int32),
        pltpu.VMEM((CH_G, G), jnp.int32),
        pltpu.VMEM((4, G), jnp.int32),
        pltpu.VMEM((4, G), jnp.int32),
        pltpu.VMEM((2, G, D), jnp.float32),
        pltpu.VMEM((NG_ACC,), jnp.float32),
        pltpu.VMEM_SHARED((NG_ACC, D), jnp.float32),
        pltpu.SemaphoreType.DMA,
        pltpu.SemaphoreType.DMA,
    ])
def _sc_gene(xd, src2, dst2, zrow, zvec, agg_out, cnt_out,
             ebuf_s, ebuf_d, ring_s, ring_d, rows, cntv, acc, sem0, sem1):
    cid = lax.axis_index("c")
    sid = lax.axis_index("s")
    iota = lax.iota(jnp.int32, L)
    zero16 = jnp.zeros((L,), jnp.int32)
    trash16 = jnp.full((L,), NG_R, jnp.int32)
    ones16 = jnp.ones((L,), jnp.float32)

    def drain(q):
        sem = sem0 if q & 1 == 0 else sem1
        pltpu.make_async_copy(xd.at[ring_s.at[q]], rows.at[q & 1], sem).wait()
        pltpu.sync_copy(rows.at[q & 1], acc.at[ring_d.at[q]], add=True)

    def start(q):
        sem = sem0 if q & 1 == 0 else sem1
        pltpu.async_copy(xd.at[ring_s.at[q]], rows.at[q & 1], sem)

    def drain4(qd):
        for q in range(4):
            @pl.when(qd == q)
            def _():
                drain(q)

    def start4(qd):
        for q in range(4):
            @pl.when(qd == q)
            def _():
                start(q)

    def pass_body(p, pcarry):
        base = (2 * p + cid) * NG_R
        pltpu.sync_copy(zrow.at[pl.ds(0, NG_ROWS)],
                        acc.at[pl.ds(sid * NG_ROWS, NG_ROWS)])
        pltpu.sync_copy(zvec.at[pl.ds(0, NG_ACC)], cntv)
        for k in range(8):
            ring_s[3, pl.ds(k * L, L)] = zero16
            ring_d[3, pl.ds(k * L, L)] = trash16
        start(3)
        plsc.subcore_barrier()

        n = jnp.int32(0)
        for ch, clen in enumerate(CHUNKS_G):
            pltpu.sync_copy(src2.at[sid, pl.ds(ch * CH_G, clen)],
                            ebuf_s.at[pl.ds(0, clen)])
            pltpu.sync_copy(dst2.at[sid, pl.ds(ch * CH_G, clen)],
                            ebuf_d.at[pl.ds(0, clen)])

            def scan_group(j, n):
                n0 = n
                for k in range(8):
                    dv = ebuf_d[j, pl.ds(k * L, L)]
                    sv = ebuf_s[j, pl.ds(k * L, L)]
                    m = (dv >= base) & (dv < base + NG_R)
                    mi = m.astype(jnp.int32)
                    loc = jnp.where(m, dv - base, NG_R)
                    plsc.addupdate_scatter(cntv, [loc], ones16, mask=m)
                    pos = n + plsc.cumsum(mi) - 1
                    plsc.store_scatter(ring_s, [(pos >> 7) & 3, pos & 127], sv, mask=m)
                    plsc.store_scatter(ring_d, [(pos >> 7) & 3, pos & 127], loc, mask=m)
                    n = n + jnp.sum(mi)

                @pl.when((n >> 7) != (n0 >> 7))
                def _():
                    g = n0 >> 7
                    drain4((g - 1) & 3)
                    start4(g & 3)

                return n

            n = lax.fori_loop(0, clen, scan_group, n)

        @pl.when((n & 127) != 0)
        def _():
            g = n >> 7
            drain4((g - 1) & 3)
            for k in range(8):
                pp = n + k * L + iota
                plsc.store_scatter(ring_d, [(pp >> 7) & 3, pp & 127], trash16)
                plsc.store_scatter(ring_s, [(pp >> 7) & 3, pp & 127], zero16)
            start4(g & 3)

        drain4((((n + G - 1) >> 7) - 1) & 3)

        plsc.subcore_barrier()
        pltpu.sync_copy(acc.at[pl.ds(sid * NG_ROWS, NG_ROWS)],
                        agg_out.at[cid, p, pl.ds(sid * NG_ROWS, NG_ROWS)])
        pltpu.sync_copy(cntv, cnt_out.at[cid, p, sid])
        plsc.subcore_barrier()
        return pcarry

    lax.fori_loop(0, 3, pass_body, 0)


@functools.partial(
    pl.kernel, mesh=_mesh,
    compiler_params=pltpu.CompilerParams(needs_layout_passes=False),
    out_type=jax.ShapeDtypeStruct((EP,), jnp.float32),
    scratch_types=[
        pltpu.VMEM((GPT, G), jnp.int32),
        pltpu.VMEM((GPT, G), jnp.int32),
        pltpu.VMEM((2, G, D), jnp.float32),
        pltpu.VMEM((2, G, D), jnp.float32),
        pltpu.VMEM((G,), jnp.float32),
        pltpu.SemaphoreType.DMA,
        pltpu.SemaphoreType.DMA,
    ])
def _sc_decoder(zd, zg, src2, dst2, out, idx_s, idx_d, rows_d, rows_g, ob,
                sem0, sem1):
    cid = lax.axis_index("c")
    sid = lax.axis_index("s")
    w = cid * NS + sid
    pltpu.sync_copy(src2.at[w], idx_s)
    pltpu.sync_copy(dst2.at[w], idx_d)
    iota = lax.iota(jnp.int32, L)
    last = iota == (L - 1)

    def start(j, b, sem):
        pltpu.async_copy(zd.at[idx_s.at[j]], rows_d.at[b], sem)
        pltpu.async_copy(zg.at[idx_d.at[j]], rows_g.at[b], sem)

    def wait(b, sem):
        pltpu.make_async_copy(zd.at[idx_s.at[0]], rows_d.at[b], sem).wait()
        pltpu.make_async_copy(zg.at[idx_d.at[0]], rows_g.at[b], sem).wait()

    def compute(j, b):
        def dot_one(e, carry):
            acc = rows_d[b, e, pl.ds(0, L)] * rows_g[b, e, pl.ds(0, L)]
            for k in range(1, 8):
                acc = acc + rows_d[b, e, pl.ds(k * L, L)] * rows_g[b, e, pl.ds(k * L, L)]
            cum = plsc.cumsum(acc)
            plsc.store_scatter(ob, [jnp.full((L,), e, jnp.int32)], cum, mask=last)
            return carry

        lax.fori_loop(0, G, dot_one, 0)
        for k in range(8):
            v = ob[pl.ds(k * L, L)]
            ob[pl.ds(k * L, L)] = 1.0 / (1.0 + jnp.exp(-v))
        pltpu.sync_copy(ob, out.at[pl.ds((w * GPT + j) * G, G)])

    start(0, 0, sem0)

    def body(j, carry):
        @pl.when(j % 2 == 0)
        def _():
            wait(0, sem0)

            @pl.when(j + 1 < GPT)
            def _():
                start(j + 1, 1, sem1)

            compute(j, 0)

        @pl.when(j % 2 == 1)
        def _():
            wait(1, sem1)

            @pl.when(j + 1 < GPT)
            def _():
                start(j + 1, 0, sem0)

            compute(j, 1)

        return carry

    lax.fori_loop(0, GPT, body, 0)


_BLK = 1000


def _dense_body(npart, *refs):
    aggs = refs[:npart]
    cnt_ref = refs[npart]
    (x_ref, wl_ref, wr_ref, b_ref, wmu_ref, bmu_ref, wlv_ref, blv_ref,
     eps_ref, z_ref) = refs[npart + 1:]
    agg = aggs[0][...]
    for a in aggs[1:]:
        agg = agg + a[...]
    cnt = jnp.sum(cnt_ref[...], axis=1, keepdims=True)
    mean = agg / jnp.maximum(cnt, 1.0)
    h = (jnp.dot(mean, wl_ref[...], preferred_element_type=jnp.float32)
         + jnp.dot(x_ref[...], wr_ref[...], preferred_element_type=jnp.float32)
         + b_ref[...][None, :])
    mu = jnp.dot(h, wmu_ref[...], preferred_element_type=jnp.float32) + bmu_ref[...][None, :]
    lv = jnp.minimum(
        jnp.dot(h, wlv_ref[...], preferred_element_type=jnp.float32) + blv_ref[...][None, :],
        10.0)
    z_ref[...] = mu + eps_ref[...] * jnp.exp(lv)


def _dense_stage(aggs, cnt, x, W_l, W_r, b, W_mu, b_mu, W_lv, b_lv, eps):
    n = x.shape[0]
    npart = len(aggs)
    ncnt = cnt.shape[1]
    row_spec = pl.BlockSpec((_BLK, D), lambda i: (i, 0))
    cnt_spec = pl.BlockSpec((_BLK, ncnt), lambda i: (i, 0))
    w_spec = pl.BlockSpec((D, DO), lambda i: (0, 0))
    b_spec = pl.BlockSpec((DO,), lambda i: (0,))
    return pl.pallas_call(
        functools.partial(_dense_body, npart),
        grid=(n // _BLK,),
        in_specs=([row_spec] * npart + [cnt_spec]
                  + [row_spec, w_spec, w_spec, b_spec, w_spec, b_spec,
                     w_spec, b_spec, row_spec]),
        out_specs=pl.BlockSpec((_BLK, DO), lambda i: (i, 0)),
        out_shape=jax.ShapeDtypeStruct((n, DO), jnp.float32),
    )(*aggs, cnt, x, W_l, W_r, b, W_mu, b_mu, W_lv, b_lv, eps)


def kernel(x_disease, x_gene, edge_src, edge_dst, W_l, W_r, b_sage,
           W_mu_d, b_mu_d, W_lv_d, b_lv_d, W_mu_g, b_mu_g, W_lv_g, b_lv_g):
    ek = jax.random.split(jax.random.key(42), 2)
    eps_d = jax.random.normal(ek[0], (ND, DO), dtype=jnp.float32)
    eps_g = jax.random.normal(ek[1], (NG, DO), dtype=jnp.float32)

    pad_e = EP - E
    srcp = jnp.concatenate([edge_src, jnp.full((pad_e,), ND, jnp.int32)])
    dstp = jnp.concatenate([edge_dst, jnp.full((pad_e,), NG, jnp.int32)])
    src32 = srcp.reshape(NW, GPT, G)
    dst32 = dstp.reshape(NW, GPT, G)
    src16 = srcp.reshape(NS, GPS, G)
    dst16 = dstp.reshape(NS, GPS, G)
    xd_pad = jnp.concatenate([x_disease, jnp.zeros((ND_PAD - ND, D), jnp.float32)])
    xg_pad = jnp.concatenate([x_gene, jnp.zeros((16, D), jnp.float32)])

    zr = max(ND_ROWS, NG_ROWS)
    zrow = jnp.zeros((zr, D), jnp.float32)
    zvec = jnp.zeros((max(ND_PAD, NG_ACC),), jnp.float32)

    agg_d_p, cnt_d_p = _sc_disease(xg_pad, src32, dst32, zrow, zvec)
    agg_g_p, cnt_g_p = _sc_gene(xd_pad, src16, dst16, zrow, zvec)

    sizes = [NG_R] * 5 + [NG - 5 * NG_R]
    agg_g = jnp.concatenate([agg_g_p[r % 2, r // 2, :sizes[r]] for r in range(6)])
    cnt_g = jnp.transpose(
        jnp.concatenate([cnt_g_p[r % 2, r // 2, :, :sizes[r]] for r in range(6)],
                        axis=1))
    cnt_d = jnp.transpose(cnt_d_p)[:ND]

    z_g = _dense_stage([agg_g], cnt_g, x_gene, W_l, W_r, b_sage,
                       W_mu_g, b_mu_g, W_lv_g, b_lv_g, eps_g)
    z_d = _dense_stage([agg_d_p[0, :ND], agg_d_p[1, :ND]],
                       cnt_d,
                       x_disease, W_l, W_r, b_sage,
                       W_mu_d, b_mu_d, W_lv_d, b_lv_d, eps_d)

    zd_pad = jnp.concatenate([z_d, jnp.zeros((ND_PAD - ND, DO), jnp.float32)])
    zg_pad = jnp.concatenate([z_g, jnp.zeros((16, DO), jnp.float32)])
    out = _sc_decoder(zd_pad, zg_pad, src32, dst32)
    return out[:E]
